# Initial kernel scaffold; baseline (speedup 1.0000x reference)
import jax
import jax.numpy as jnp
from jax.experimental import pallas as pl


def kernel(h, pos, Wmlp1, bmlp1, Wmlp2, bmlp2, Wl1, Wl2, bl2, Wlin, blin):
    raise NotImplementedError("write your pallas kernel here")


# speedup vs baseline: 1.1041x; 1.1041x over previous
import numpy as np
import jax
import jax.numpy as jnp
from jax.experimental import pallas as pl
from jax.experimental.pallas import tpu as pltpu

_N = 512
_HIDDEN = 128
_FILTERS = 128
_GAUSS = 50
_CUTOFF = 10.0
_TI = 128
_TJ = 128
_NI = _N // _TI
_NJ = _N // _TJ

_OFFSET = np.linspace(0.0, _CUTOFF, _GAUSS).astype(np.float32)
_COEFF = np.float32(-0.5 / (_OFFSET[1] - _OFFSET[0]) ** 2)
_LOG2 = np.float32(np.log(2.0))


def _ssp_stable(x):
    return jnp.maximum(x, 0.0) + jnp.log1p(jnp.exp(-jnp.abs(x))) - _LOG2


def _schnet_kernel(h_ref, pos_ref, posT_ref, off_ref,
                   wm1_ref, bm1_ref, wm2_ref, bm2_ref,
                   wl1_ref, wl2_ref, bl2_ref, wlin_ref, blin_ref,
                   out_ref, acc_ref, x1_ref):
    i = pl.program_id(0)
    j = pl.program_id(1)

    @pl.when(i == 0)
    def _():
        hj = h_ref[pl.ds(j * _TJ, _TJ), :]
        x1_ref[pl.ds(j * _TJ, _TJ), :] = jnp.dot(
            hj, wl1_ref[:, :], preferred_element_type=jnp.float32)

    pi = pos_ref[pl.ds(i * _TI, _TI), :]
    pjT = posT_ref[:, pl.ds(j * _TJ, _TJ)]
    dx = pi[:, 0:1] - pjT[0:1, :]
    dy = pi[:, 1:2] - pjT[1:2, :]
    dz = pi[:, 2:3] - pjT[2:3, :]
    d2 = dx * dx + dy * dy + dz * dz
    safe = jnp.where(d2 > 0.0, d2, 1.0)
    dist = jnp.where(d2 > 0.0, jnp.sqrt(safe), 0.0)

    rows = jax.lax.broadcasted_iota(jnp.int32, (_TI, _TJ), 0) + i * _TI
    cols = jax.lax.broadcasted_iota(jnp.int32, (_TI, _TJ), 1) + j * _TJ
    cw = 0.5 * (jnp.cos(dist * (np.pi / _CUTOFF)) + 1.0)
    keep = (dist < _CUTOFF) & (rows != cols)
    scale = jnp.where(keep, cw, 0.0)

    d3 = dist.reshape(_TI, _TJ, 1)
    off3 = off_ref[0:1, :].reshape(1, 1, _GAUSS)
    delta = d3 - off3
    ea = jnp.exp(_COEFF * (delta * delta))
    ea2 = ea.reshape(_TI * _TJ, _GAUSS)

    t1 = jnp.dot(ea2, wm1_ref[:, :],
                 preferred_element_type=jnp.float32) + bm1_ref[0:1, :]
    a1 = jnp.log1p(jnp.exp(t1)) - _LOG2
    wf = jnp.dot(a1, wm2_ref[:, :],
                 preferred_element_type=jnp.float32) + bm2_ref[0:1, :]

    wf3 = wf.reshape(_TI, _TJ, _FILTERS) * scale.reshape(_TI, _TJ, 1)
    x1j = x1_ref[pl.ds(j * _TJ, _TJ), :]
    contrib = jnp.sum(wf3 * x1j[None, :, :], axis=1)

    @pl.when(j == 0)
    def _():
        acc_ref[:, :] = contrib

    @pl.when(j > 0)
    def _():
        acc_ref[:, :] = acc_ref[:, :] + contrib

    @pl.when(j == _NJ - 1)
    def _():
        x2 = jnp.dot(acc_ref[:, :], wl2_ref[:, :],
                     preferred_element_type=jnp.float32) + bl2_ref[0:1, :]
        x3 = _ssp_stable(x2)
        out_ref[:, :] = jnp.dot(x3, wlin_ref[:, :],
                                preferred_element_type=jnp.float32) + blin_ref[0:1, :]


def _full(shape):
    return pl.BlockSpec(shape, lambda i, j: tuple(0 for _ in shape))


@jax.jit
def kernel(h, pos, Wmlp1, bmlp1, Wmlp2, bmlp2, Wl1, Wl2, bl2, Wlin, blin):
    posT = pos.T
    off = jnp.asarray(_OFFSET).reshape(1, _GAUSS)
    args = (h, pos, posT, off,
            Wmlp1, bmlp1.reshape(1, -1), Wmlp2, bmlp2.reshape(1, -1),
            Wl1, Wl2, bl2.reshape(1, -1), Wlin, blin.reshape(1, -1))
    return pl.pallas_call(
        _schnet_kernel,
        grid=(_NI, _NJ),
        in_specs=[_full(a.shape) for a in args],
        out_specs=pl.BlockSpec((_TI, _HIDDEN), lambda i, j: (i, 0)),
        out_shape=jax.ShapeDtypeStruct((_N, _HIDDEN), jnp.float32),
        scratch_shapes=[
            pltpu.VMEM((_TI, _FILTERS), jnp.float32),
            pltpu.VMEM((_N, _FILTERS), jnp.float32),
        ],
    )(*args)
